# baseline (speedup 1.0000x reference)
import functools

import jax
import jax.numpy as jnp
from jax import lax
from jax.experimental import pallas as pl
from jax.experimental.pallas import tpu as pltpu
from jax.experimental.pallas import tpu_sc as plsc

B = 16384
D = 32
NC, NS = 2, 16
NW = NC * NS
BPW = B // NW
CHUNK = 128
NCHUNK = BPW // CHUNK

def _sc_gather_body(uidx_hbm, iidx_hbm, user_hbm, item_hbm,
                    ue_hbm, ie_hbm,
                    uidx_v, iidx_v, urows_v, irows_v, sem):
    wid = lax.axis_index("s") * NC + lax.axis_index("c")
    base = wid * BPW
    pltpu.sync_copy(uidx_hbm.at[wid], uidx_v)
    pltpu.sync_copy(iidx_hbm.at[wid], iidx_v)
    copies = []
    for j in range(NCHUNK):
        copies.append(pltpu.async_copy(
            user_hbm.at[uidx_v.at[j]], urows_v.at[pl.ds(j * CHUNK, CHUNK)], sem))
        copies.append(pltpu.async_copy(
            item_hbm.at[iidx_v.at[j]], irows_v.at[pl.ds(j * CHUNK, CHUNK)], sem))
    for c in copies:
        c.wait()
    pltpu.sync_copy(urows_v, ue_hbm.at[pl.ds(base, BPW)])
    pltpu.sync_copy(irows_v, ie_hbm.at[pl.ds(base, BPW)])


@functools.cache
def _sc_gather():
    mesh = plsc.VectorSubcoreMesh(
        core_axis_name="c", subcore_axis_name="s", num_cores=NC, num_subcores=NS
    )
    return pl.kernel(
        _sc_gather_body,
        out_type=(
            jax.ShapeDtypeStruct((B, D), jnp.float32),
            jax.ShapeDtypeStruct((B, D), jnp.float32),
        ),
        mesh=mesh,
        scratch_types=[
            pltpu.VMEM((NCHUNK, CHUNK), jnp.int32),
            pltpu.VMEM((NCHUNK, CHUNK), jnp.int32),
            pltpu.VMEM((BPW, D), jnp.float32),
            pltpu.VMEM((BPW, D), jnp.float32),
            pltpu.SemaphoreType.DMA,
        ],
        compiler_params=pltpu.CompilerParams(use_tc_tiling_on_sc=False),
    )


def _mlp_body(ue_ref, ie_ref, wu_ref, wi_ref, bc_ref, wb_ref, out_ref):
    h = jnp.dot(ue_ref[...], wu_ref[...], preferred_element_type=jnp.float32)
    h = h + jnp.dot(ie_ref[...], wi_ref[...], preferred_element_type=jnp.float32)
    h = jnp.maximum(h + bc_ref[...], 0.0)
    out_ref[...] = jnp.dot(h, wb_ref[...], preferred_element_type=jnp.float32)


_BS = 2048


def _mlp(ue, ie, wu, wi, bc, wb):
    return pl.pallas_call(
        _mlp_body,
        grid=(B // _BS,),
        in_specs=[
            pl.BlockSpec((_BS, D), lambda i: (i, 0)),
            pl.BlockSpec((_BS, D), lambda i: (i, 0)),
            pl.BlockSpec((D, 2 * D), lambda i: (0, 0)),
            pl.BlockSpec((D, 2 * D), lambda i: (0, 0)),
            pl.BlockSpec((1, 2 * D), lambda i: (0, 0)),
            pl.BlockSpec((2 * D, 2), lambda i: (0, 0)),
        ],
        out_specs=pl.BlockSpec((_BS, 2), lambda i: (i, 0)),
        out_shape=jax.ShapeDtypeStruct((B, 2), jnp.float32),
    )(ue, ie, wu, wi, bc, wb)


def kernel(x, user_emb, item_emb, W1a, b1a, W1b, W0a, b0a, W0b):
    uidx = x[:, 0].astype(jnp.int32).reshape(NW, NCHUNK, CHUNK)
    iidx = x[:, 1].astype(jnp.int32).reshape(NW, NCHUNK, CHUNK)
    ue, ie = _sc_gather()(uidx, iidx, user_emb, item_emb)

    wu = jnp.concatenate([W1a[:, :D], W0a[:, :D]], axis=0).T
    wi = jnp.concatenate([W1a[:, D:], W0a[:, D:]], axis=0).T
    bc = jnp.concatenate([b1a, b0a]).reshape(1, 2 * D)
    wb = jnp.zeros((2 * D, 2), jnp.float32)
    wb = wb.at[:D, 0].set(W1b[0]).at[D:, 1].set(W0b[0])

    y = _mlp(ue, ie, wu, wi, bc, wb)
    return (y[:, 0:1], y[:, 1:2])
